# baseline (speedup 1.0000x reference)
import functools

import jax
import jax.numpy as jnp
from jax import lax
from jax.experimental import pallas as pl
from jax.experimental.pallas import tpu as pltpu
from jax.experimental.pallas import tpu_sc as plsc

N = 10000
C = 500
ORIG = 128
ATOM = 48
HID = 16
H = 128

BLK = 1024
NP = 10240
NBLK = NP // BLK
CP = 512
WIDE = 64

NW = 32
ROWS_W = NP // NW
CH = 80
NCH = ROWS_W // CH

_EPS_BN = 1e-5
_EPS_DEN = 1e-13


def _tc1_body(x_ref, aw_ref, idxc_ref,
              wemb_ref, bemb_ref, wg1_ref, bg1_ref, ggam_ref, gbet_ref,
              wg2_ref, bg2_ref, eaf_ref, af_ref):
    f32 = jnp.float32
    wemb = wemb_ref[:, :]
    bemb = bemb_ref[:, :]
    wg1 = wg1_ref[:, :]
    bg1 = bg1_ref[:, :]
    wg2 = wg2_ref[:, :]
    bg2 = bg2_ref[:, :]

    def loop_a(t, carry):
        s1, s2 = carry
        xb = x_ref[pl.ds(t * BLK, BLK), :]
        af = jnp.dot(xb, wemb, preferred_element_type=f32) + bemb
        af_ref[pl.ds(t * BLK, BLK), :] = af
        z = jnp.dot(af, wg1, preferred_element_type=f32) + bg1
        rmask = (jax.lax.broadcasted_iota(jnp.int32, (BLK, 1), 0)
                 + t * BLK < N).astype(f32)
        zm = z * rmask
        s1 = s1 + jnp.sum(zm, axis=0, keepdims=True)
        s2 = s2 + jnp.sum(zm * z, axis=0, keepdims=True)
        return s1, s2

    s1, s2 = jax.lax.fori_loop(
        0, NBLK, loop_a,
        (jnp.zeros((1, HID), f32), jnp.zeros((1, HID), f32)))
    mean = s1 / N
    var = s2 / N - mean * mean
    scale = ggam_ref[:, :] * jax.lax.rsqrt(var + _EPS_BN)
    shift = gbet_ref[:, :] - mean * scale

    def gate_block(t):
        af = af_ref[pl.ds(t * BLK, BLK), :]
        z = jnp.dot(af, wg1, preferred_element_type=f32) + bg1
        h = jnp.maximum(z * scale + shift, 0.0)
        g = jnp.dot(h, wg2, preferred_element_type=f32) + bg2
        return af, g

    ciota_row = jax.lax.broadcasted_iota(jnp.int32, (1, CP), 1)
    neg_inf = jnp.float32(-jnp.inf)

    def loop_b(t, smax):
        _, g = gate_block(t)
        oh = idxc_ref[pl.ds(t * BLK, BLK), :] == ciota_row
        vals = jnp.where(oh, g, neg_inf)
        return jnp.maximum(smax, jnp.max(vals, axis=0, keepdims=True))

    smax = jax.lax.fori_loop(0, NBLK, loop_b,
                             jnp.full((1, CP), neg_inf, f32))

    def loop_c(t, _):
        af, g = gate_block(t)
        oh = idxc_ref[pl.ds(t * BLK, BLK), :] == ciota_row
        gathered = jnp.max(jnp.where(oh, smax, neg_inf), axis=1,
                           keepdims=True)
        e = aw_ref[pl.ds(t * BLK, BLK), :] * jnp.exp(g - gathered)
        eaf_ref[pl.ds(t * BLK, BLK), 0:ATOM] = af * e
        eaf_ref[pl.ds(t * BLK, BLK), ATOM:ATOM + 1] = e
        eaf_ref[pl.ds(t * BLK, BLK), ATOM + 1:WIDE] = jnp.zeros(
            (BLK, WIDE - ATOM - 1), f32)
        return 0

    jax.lax.fori_loop(0, NBLK, loop_c, 0)


def _sc_scatter(eaf_hbm, idx_hbm, zeros_hbm, out_hbm, idx_v, rows_v, acc_v):
    ci = lax.axis_index("c")
    si = lax.axis_index("s")
    wid = si * 2 + ci
    base = wid * ROWS_W
    pltpu.sync_copy(zeros_hbm, acc_v)
    pltpu.sync_copy(idx_hbm.at[pl.ds(base, ROWS_W)], idx_v)
    pltpu.sync_copy(eaf_hbm.at[pl.ds(base, ROWS_W)], rows_v)

    def grp(i, carry):
        vec = idx_v[pl.ds(i * 16, 16)]
        for l in range(16):
            s = vec[l]
            j = i * 16 + l
            for k in range(WIDE // 16):
                plsc.addupdate(acc_v.at[s, pl.ds(k * 16, 16)],
                               rows_v[j, pl.ds(k * 16, 16)])
        return carry

    jax.lax.fori_loop(0, ROWS_W // 16, grp, 0)
    pltpu.sync_copy(acc_v, out_hbm.at[wid])


def _tc2_body(parts_ref, wfc_ref, bfc_ref, fgam_ref, fbet_ref,
              wout_ref, bout_ref, out_ref):
    f32 = jnp.float32

    def loop_sum(i, a):
        return a + parts_ref[i]

    acc = jax.lax.fori_loop(0, NW, loop_sum,
                            jnp.zeros((CP, WIDE), f32))
    crys = acc[:, :ATOM] / (acc[:, ATOM:ATOM + 1] + _EPS_DEN)
    y = jnp.dot(crys, wfc_ref[:, :], preferred_element_type=f32) + bfc_ref[:, :]
    ciota_col = jax.lax.broadcasted_iota(jnp.int32, (CP, 1), 0)
    cmask = (ciota_col < C).astype(f32)
    ym = y * cmask
    m2 = jnp.sum(ym, axis=0, keepdims=True) / C
    v2 = jnp.sum(ym * y, axis=0, keepdims=True) / C - m2 * m2
    yn = (y - m2) * (fgam_ref[:, :] * jax.lax.rsqrt(v2 + _EPS_BN)) + fbet_ref[:, :]
    sp = jnp.maximum(yn, 0.0) + jnp.log1p(jnp.exp(-jnp.abs(yn)))
    out_ref[:, :] = jnp.dot(sp, wout_ref[:, :],
                            preferred_element_type=f32) + bout_ref[:, :]


@jax.jit
def kernel(atom_weights, orig_atom_fea, nbr_fea, self_fea_idx, nbr_fea_idx,
           crystal_atom_idx, W_emb, b_emb, W_g1, b_g1, g_gamma, g_beta,
           W_g2, b_g2, W_fc, b_fc, fc_gamma, fc_beta, W_out, b_out):
    del nbr_fea, self_fea_idx, nbr_fea_idx
    f32 = jnp.float32
    pad = NP - N
    xp = jnp.pad(orig_atom_fea, ((0, pad), (0, 0)))
    awp = jnp.pad(atom_weights, ((0, pad), (0, 0)))
    idx = crystal_atom_idx.astype(jnp.int32)
    idxp = jnp.pad(idx, (0, pad), constant_values=CP - 1)
    idx_col = idxp.reshape(NP, 1)

    eafw = pl.pallas_call(
        _tc1_body,
        out_shape=jax.ShapeDtypeStruct((NP, WIDE), f32),
        scratch_shapes=[pltpu.VMEM((NP, ATOM), f32)],
    )(xp, awp, idx_col,
      W_emb, b_emb.reshape(1, ATOM), W_g1, b_g1.reshape(1, HID),
      g_gamma.reshape(1, HID), g_beta.reshape(1, HID),
      W_g2, b_g2.reshape(1, 1))

    sc_call = functools.partial(
        pl.kernel,
        out_type=jax.ShapeDtypeStruct((NW, CP, WIDE), f32),
        mesh=plsc.VectorSubcoreMesh(core_axis_name="c", subcore_axis_name="s"),
        scratch_types=[
            pltpu.VMEM((ROWS_W,), jnp.int32),
            pltpu.VMEM((ROWS_W, WIDE), f32),
            pltpu.VMEM((CP, WIDE), f32),
        ],
    )(_sc_scatter)
    parts = sc_call(eafw, idxp, jnp.zeros((CP, WIDE), f32))

    out = pl.pallas_call(
        _tc2_body,
        out_shape=jax.ShapeDtypeStruct((CP, 1), f32),
    )(parts, W_fc, b_fc.reshape(1, H), fc_gamma.reshape(1, H),
      fc_beta.reshape(1, H), W_out, b_out.reshape(1, 1))
    return out[:C]


# speedup vs baseline: 1.0041x; 1.0041x over previous
import functools

import jax
import jax.numpy as jnp
from jax import lax
from jax.experimental import pallas as pl
from jax.experimental.pallas import tpu as pltpu
from jax.experimental.pallas import tpu_sc as plsc

N = 10000
C = 500
ORIG = 128
ATOM = 48
HID = 16
H = 128

BLK = 1024
NP = 10240
NBLK = NP // BLK
CP = 512
WIDE = 64

NW = 32
ROWS_W = NP // NW
ROWS_T = NP // 16
LANES = 16
STRIPE = CP // 16

_EPS_BN = 1e-5
_EPS_DEN = 1e-13


def _tc1_body(x_ref, wemb_ref, bemb_ref, wg1_ref, bg1_ref, ggam_ref,
              gbet_ref, wg2_ref, bg2_ref, af_ref, g_ref):
    f32 = jnp.float32
    wemb = wemb_ref[:, :]
    bemb = bemb_ref[:, :]
    wg1 = wg1_ref[:, :]
    bg1 = bg1_ref[:, :]

    def loop_a(t, carry):
        s1, s2 = carry
        xb = x_ref[pl.ds(t * BLK, BLK), :]
        af = jnp.dot(xb, wemb, preferred_element_type=f32) + bemb
        af_ref[pl.ds(t * BLK, BLK), :] = af
        z = jnp.dot(af, wg1, preferred_element_type=f32) + bg1
        rmask = (jax.lax.broadcasted_iota(jnp.int32, (BLK, 1), 0)
                 + t * BLK < N).astype(f32)
        zm = z * rmask
        s1 = s1 + jnp.sum(zm, axis=0, keepdims=True)
        s2 = s2 + jnp.sum(zm * z, axis=0, keepdims=True)
        return s1, s2

    s1, s2 = jax.lax.fori_loop(
        0, NBLK, loop_a,
        (jnp.zeros((1, HID), f32), jnp.zeros((1, HID), f32)))
    mean = s1 / N
    var = s2 / N - mean * mean
    scale = ggam_ref[:, :] * jax.lax.rsqrt(var + _EPS_BN)
    shift = gbet_ref[:, :] - mean * scale

    def loop_g(t, carry):
        af = af_ref[pl.ds(t * BLK, BLK), :]
        z = jnp.dot(af, wg1, preferred_element_type=f32) + bg1
        h = jnp.maximum(z * scale + shift, 0.0)
        g = jnp.dot(h, wg2_ref[:, :], preferred_element_type=f32) + bg2_ref[:, :]
        g_ref[pl.ds(t * BLK, BLK), :] = g
        return carry

    jax.lax.fori_loop(0, NBLK, loop_g, 0)


SLABW = CP * LANES
ACCW = CP * WIDE
STW = STRIPE * LANES
STA = STRIPE * WIDE


def _sc_body(af_hbm, g_hbm, aw_hbm, idx_hbm, zeros_hbm, neginf_hbm, out_hbm,
             idx1_v, g1_v, maxacc_v, stripe_tmp, stripe_max, smax_v,
             idx2_v, g2_v, aw2_v, af_v, stage_v, acc_v, acc_tmp, acc_sum,
             sp_slabs, sp_smax, sp_acc):
    f32 = jnp.float32
    ci = lax.axis_index("c")
    si = lax.axis_index("s")
    wid = si * 2 + ci

    base1 = si * ROWS_T
    pltpu.sync_copy(neginf_hbm, maxacc_v)
    pltpu.sync_copy(idx_hbm.at[pl.ds(base1, ROWS_T)], idx1_v)
    pltpu.sync_copy(g_hbm.at[pl.ds(base1, ROWS_T)], g1_v)
    cur0 = idx1_v[pl.ds(0, LANES)][0]
    neg_inf = jnp.float32(-jnp.inf)

    def grp1(i, carry):
        cur, m = carry
        idx_vec = idx1_v[pl.ds(i * LANES, LANES)]
        g_vec = g1_v[pl.ds(i * LANES, LANES)]
        for l in range(LANES):
            s = idx_vec[l]
            gv = g_vec[l]
            flush = s != cur

            @pl.when(flush)
            def _(cur=cur, m=m):
                maxacc_v[pl.ds(cur * LANES, LANES)] = jnp.full((LANES,), m, f32)

            m = jnp.where(flush, gv, jnp.maximum(m, gv))
            cur = jnp.where(flush, s, cur)
        return cur, m

    cur, m = jax.lax.fori_loop(0, ROWS_T // LANES, grp1, (cur0, neg_inf))
    maxacc_v[pl.ds(cur * LANES, LANES)] = jnp.full((LANES,), m, f32)

    pltpu.sync_copy(maxacc_v, sp_slabs.at[si])
    plsc.subcore_barrier()
    pltpu.sync_copy(sp_slabs.at[0, pl.ds(si * STW, STW)], stripe_max)

    def merge_max(t, carry):
        pltpu.sync_copy(sp_slabs.at[t, pl.ds(si * STW, STW)], stripe_tmp)
        for r in range(STW // LANES):
            stripe_max[pl.ds(r * LANES, LANES)] = jnp.maximum(
                stripe_max[pl.ds(r * LANES, LANES)],
                stripe_tmp[pl.ds(r * LANES, LANES)])
        return carry

    jax.lax.fori_loop(1, 16, merge_max, 0)
    pltpu.sync_copy(stripe_max, sp_smax.at[pl.ds(si * STW, STW)])
    plsc.subcore_barrier()
    pltpu.sync_copy(sp_smax, smax_v)

    base2 = wid * ROWS_W
    pltpu.sync_copy(zeros_hbm, acc_v)
    pltpu.sync_copy(idx_hbm.at[pl.ds(base2, ROWS_W)], idx2_v)
    pltpu.sync_copy(g_hbm.at[pl.ds(base2, ROWS_W)], g2_v)
    pltpu.sync_copy(aw_hbm.at[pl.ds(base2, ROWS_W)], aw2_v)
    pltpu.sync_copy(af_hbm.at[pl.ds(base2 * ATOM, ROWS_W * ATOM)], af_v)
    cur0b = idx2_v[pl.ds(0, LANES)][0]
    zvec = jnp.zeros((LANES,), f32)
    for k in range(WIDE // LANES):
        stage_v[pl.ds(k * LANES, LANES)] = zvec

    def grp2(i, cur):
        b16 = i * LANES
        idx_vec = idx2_v[pl.ds(b16, LANES)]
        g_vec = g2_v[pl.ds(b16, LANES)]
        aw_vec = aw2_v[pl.ds(b16, LANES)]
        a0 = stage_v[pl.ds(0, LANES)]
        a1 = stage_v[pl.ds(LANES, LANES)]
        a2 = stage_v[pl.ds(2 * LANES, LANES)]
        a3 = stage_v[pl.ds(3 * LANES, LANES)]
        for l in range(LANES):
            s = idx_vec[l]
            j = b16 + l
            gs = jnp.full((LANES,), g_vec[l], f32)
            aws = jnp.full((LANES,), aw_vec[l], f32)
            ms = smax_v[pl.ds(s * LANES, LANES)]
            es = aws * jnp.exp(gs - ms)
            r0 = af_v[pl.ds(j * ATOM, LANES)] * es
            r1 = af_v[pl.ds(j * ATOM + LANES, LANES)] * es
            r2 = af_v[pl.ds(j * ATOM + 2 * LANES, LANES)] * es
            flush = s != cur

            @pl.when(flush)
            def _(cur=cur, a0=a0, a1=a1, a2=a2, a3=a3):
                acc_v[pl.ds(cur * WIDE, LANES)] = a0
                acc_v[pl.ds(cur * WIDE + LANES, LANES)] = a1
                acc_v[pl.ds(cur * WIDE + 2 * LANES, LANES)] = a2
                acc_v[pl.ds(cur * WIDE + 3 * LANES, LANES)] = a3

            kf = jnp.where(flush, jnp.float32(0.0), jnp.float32(1.0))
            a0 = r0 + a0 * kf
            a1 = r1 + a1 * kf
            a2 = r2 + a2 * kf
            a3 = es + a3 * kf
            cur = jnp.where(flush, s, cur)
        stage_v[pl.ds(0, LANES)] = a0
        stage_v[pl.ds(LANES, LANES)] = a1
        stage_v[pl.ds(2 * LANES, LANES)] = a2
        stage_v[pl.ds(3 * LANES, LANES)] = a3
        return cur

    cur = jax.lax.fori_loop(0, ROWS_W // LANES, grp2, cur0b)
    acc_v[pl.ds(cur * WIDE, LANES)] = stage_v[pl.ds(0, LANES)]
    acc_v[pl.ds(cur * WIDE + LANES, LANES)] = stage_v[pl.ds(LANES, LANES)]
    acc_v[pl.ds(cur * WIDE + 2 * LANES, LANES)] = stage_v[pl.ds(2 * LANES, LANES)]
    acc_v[pl.ds(cur * WIDE + 3 * LANES, LANES)] = stage_v[pl.ds(3 * LANES, LANES)]

    pltpu.sync_copy(acc_v, sp_acc.at[si])
    plsc.subcore_barrier()
    pltpu.sync_copy(sp_acc.at[0, pl.ds(si * STA, STA)], acc_sum)

    def merge_sum(t, carry):
        pltpu.sync_copy(sp_acc.at[t, pl.ds(si * STA, STA)], acc_tmp)
        for r in range(STA // LANES):
            acc_sum[pl.ds(r * LANES, LANES)] = (
                acc_sum[pl.ds(r * LANES, LANES)]
                + acc_tmp[pl.ds(r * LANES, LANES)])
        return carry

    jax.lax.fori_loop(1, 16, merge_sum, 0)
    pltpu.sync_copy(acc_sum, out_hbm.at[pl.ds(ci * ACCW + si * STA, STA)])


def _tc2_body(parts_ref, wfc_ref, bfc_ref, fgam_ref, fbet_ref,
              wout_ref, bout_ref, out_ref):
    f32 = jnp.float32

    acc = parts_ref[0:CP, :] + parts_ref[CP:2 * CP, :]
    crys = acc[:, :ATOM] / (acc[:, ATOM:ATOM + 1] + _EPS_DEN)
    y = jnp.dot(crys, wfc_ref[:, :], preferred_element_type=f32) + bfc_ref[:, :]
    ciota_col = jax.lax.broadcasted_iota(jnp.int32, (CP, 1), 0)
    cmask = (ciota_col < C).astype(f32)
    ym = y * cmask
    m2 = jnp.sum(ym, axis=0, keepdims=True) / C
    v2 = jnp.sum(ym * y, axis=0, keepdims=True) / C - m2 * m2
    yn = (y - m2) * (fgam_ref[:, :] * jax.lax.rsqrt(v2 + _EPS_BN)) + fbet_ref[:, :]
    sp = jnp.maximum(yn, 0.0) + jnp.log1p(jnp.exp(-jnp.abs(yn)))
    out_ref[:, :] = jnp.dot(sp, wout_ref[:, :],
                            preferred_element_type=f32) + bout_ref[:, :]


@jax.jit
def kernel(atom_weights, orig_atom_fea, nbr_fea, self_fea_idx, nbr_fea_idx,
           crystal_atom_idx, W_emb, b_emb, W_g1, b_g1, g_gamma, g_beta,
           W_g2, b_g2, W_fc, b_fc, fc_gamma, fc_beta, W_out, b_out):
    del nbr_fea, self_fea_idx, nbr_fea_idx
    f32 = jnp.float32
    pad = NP - N
    xp = jnp.pad(orig_atom_fea, ((0, pad), (0, 0)))
    awp = jnp.pad(atom_weights, ((0, pad), (0, 0))).reshape(NP)
    idx = crystal_atom_idx.astype(jnp.int32)
    idxp = jnp.pad(idx, (0, pad), constant_values=CP - 1)

    af, g = pl.pallas_call(
        _tc1_body,
        out_shape=[jax.ShapeDtypeStruct((NP, ATOM), f32),
                   jax.ShapeDtypeStruct((NP, 1), f32)],
    )(xp, W_emb, b_emb.reshape(1, ATOM), W_g1, b_g1.reshape(1, HID),
      g_gamma.reshape(1, HID), g_beta.reshape(1, HID),
      W_g2, b_g2.reshape(1, 1))

    sc_call = functools.partial(
        pl.kernel,
        out_type=jax.ShapeDtypeStruct((2 * ACCW,), f32),
        mesh=plsc.VectorSubcoreMesh(core_axis_name="c", subcore_axis_name="s"),
        scratch_types=[
            pltpu.VMEM((ROWS_T,), jnp.int32),
            pltpu.VMEM((ROWS_T,), f32),
            pltpu.VMEM((SLABW,), f32),
            pltpu.VMEM((STW,), f32),
            pltpu.VMEM((STW,), f32),
            pltpu.VMEM((SLABW,), f32),
            pltpu.VMEM((ROWS_W,), jnp.int32),
            pltpu.VMEM((ROWS_W,), f32),
            pltpu.VMEM((ROWS_W,), f32),
            pltpu.VMEM((ROWS_W * ATOM,), f32),
            pltpu.VMEM((WIDE,), f32),
            pltpu.VMEM((ACCW,), f32),
            pltpu.VMEM((STA,), f32),
            pltpu.VMEM((STA,), f32),
            pltpu.VMEM_SHARED((16, SLABW), f32),
            pltpu.VMEM_SHARED((SLABW,), f32),
            pltpu.VMEM_SHARED((16, ACCW), f32),
        ],
    )(_sc_body)
    parts = sc_call(af.reshape(NP * ATOM), g.reshape(NP), awp, idxp,
                    jnp.zeros((ACCW,), f32),
                    jnp.full((SLABW,), -jnp.inf, f32))

    out = pl.pallas_call(
        _tc2_body,
        out_shape=jax.ShapeDtypeStruct((CP, 1), f32),
    )(parts.reshape(2 * CP, WIDE), W_fc, b_fc.reshape(1, H),
      fc_gamma.reshape(1, H), fc_beta.reshape(1, H), W_out,
      b_out.reshape(1, 1))
    return out[:C]
